# baseline (speedup 1.0000x reference)
import math

import jax
import jax.numpy as jnp
from jax.experimental import pallas as pl
from jax.experimental.pallas import tpu as pltpu

_C0 = math.sqrt(2.0 / math.pi)
_C1 = 0.044715


def _gelu_block(x_ref, o_ref):
    x = x_ref[...]
    inner = _C0 * (x + _C1 * (x * x * x))
    o_ref[...] = 0.5 * x * (1.0 + jnp.tanh(inner))


def kernel(x, log_tau, log_blend):
    b, t, d = x.shape
    rows = b * t
    x2 = x.reshape(rows, d)
    block_rows = 256
    out = pl.pallas_call(
        _gelu_block,
        grid=(rows // block_rows,),
        in_specs=[pl.BlockSpec((block_rows, d), lambda i: (i, 0))],
        out_specs=pl.BlockSpec((block_rows, d), lambda i: (i, 0)),
        out_shape=jax.ShapeDtypeStruct((rows, d), x.dtype),
        compiler_params=pltpu.CompilerParams(
            dimension_semantics=("arbitrary",),
        ),
    )(x2)
    return out.reshape(b, t, d)


# speedup vs baseline: 1.2055x; 1.2055x over previous
import math

import jax
import jax.numpy as jnp
from jax.experimental import pallas as pl
from jax.experimental.pallas import tpu as pltpu

_C0 = math.sqrt(2.0 / math.pi)
_C1 = 0.044715


def _gelu_block(x_ref, o_ref):
    x = x_ref[...]
    inner = _C0 * (x + _C1 * (x * x * x))
    o_ref[...] = 0.5 * x * (1.0 + jnp.tanh(inner))


def kernel(x, log_tau, log_blend):
    b, t, d = x.shape
    rows = b * t
    x2 = x.reshape(rows, d)
    block_rows = 1024
    out = pl.pallas_call(
        _gelu_block,
        grid=(rows // block_rows,),
        in_specs=[pl.BlockSpec((block_rows, d), lambda i: (i, 0))],
        out_specs=pl.BlockSpec((block_rows, d), lambda i: (i, 0)),
        out_shape=jax.ShapeDtypeStruct((rows, d), x.dtype),
        compiler_params=pltpu.CompilerParams(
            dimension_semantics=("arbitrary",),
        ),
    )(x2)
    return out.reshape(b, t, d)


# speedup vs baseline: 1.2484x; 1.0356x over previous
import math

import jax
import jax.numpy as jnp
from jax.experimental import pallas as pl
from jax.experimental.pallas import tpu as pltpu

_C0 = math.sqrt(2.0 / math.pi)
_C1 = 0.044715
_B1 = _C0 * _C1

CHUNK = 512
NBUF = 3


def _gelu(x):
    t = x * x
    u = _B1 * t + _C0
    th = jnp.tanh(x * u)
    h = 0.5 * x
    return h * th + h


def _pipeline(x_hbm, o_hbm, xbuf, obuf, in_sem, out_sem):
    nchunks = x_hbm.shape[0] // CHUNK

    def get(i, slot):
        return pltpu.make_async_copy(
            x_hbm.at[pl.ds(i * CHUNK, CHUNK), :], xbuf.at[slot], in_sem.at[slot]
        )

    def put(i, slot):
        return pltpu.make_async_copy(
            obuf.at[slot], o_hbm.at[pl.ds(i * CHUNK, CHUNK), :], out_sem.at[slot]
        )

    for k in range(NBUF):
        get(k, k).start()

    def step(i, _):
        slot = jax.lax.rem(i, NBUF)
        get(i, slot).wait()

        @pl.when(i >= NBUF)
        def _():
            put(i - NBUF, slot).wait()

        obuf[slot] = _gelu(xbuf[slot])
        put(i, slot).start()

        @pl.when(i + NBUF < nchunks)
        def _():
            get(i + NBUF, slot).start()

        return 0

    jax.lax.fori_loop(0, nchunks, step, 0)

    for k in range(NBUF):
        last = nchunks - NBUF + k
        put(last, jax.lax.rem(jnp.int32(last), NBUF)).wait()


def kernel(x, log_tau, log_blend):
    b, t, d = x.shape
    rows = b * t
    x2 = x.reshape(rows, d)
    out = pl.pallas_call(
        _pipeline,
        in_specs=[pl.BlockSpec(memory_space=pl.ANY)],
        out_specs=pl.BlockSpec(memory_space=pl.ANY),
        out_shape=jax.ShapeDtypeStruct((rows, d), x.dtype),
        scratch_shapes=[
            pltpu.VMEM((NBUF, CHUNK, d), jnp.float32),
            pltpu.VMEM((NBUF, CHUNK, d), jnp.float32),
            pltpu.SemaphoreType.DMA((NBUF,)),
            pltpu.SemaphoreType.DMA((NBUF,)),
        ],
    )(x2)
    return out.reshape(b, t, d)


# speedup vs baseline: 1.2496x; 1.0010x over previous
import math

import jax
import jax.numpy as jnp
from jax.experimental import pallas as pl
from jax.experimental.pallas import tpu as pltpu

_C0 = math.sqrt(2.0 / math.pi)
_C1 = 0.044715
_B1 = _C0 * _C1

CHUNK = 256
NBUF = 6


def _gelu(x):
    t = x * x
    u = _B1 * t + _C0
    th = jnp.tanh(x * u)
    h = 0.5 * x
    return h * th + h


def _pipeline(x_hbm, o_hbm, xbuf, obuf, in_sem, out_sem):
    nchunks = x_hbm.shape[0] // CHUNK

    def get(i, slot):
        return pltpu.make_async_copy(
            x_hbm.at[pl.ds(i * CHUNK, CHUNK), :], xbuf.at[slot], in_sem.at[slot]
        )

    def put(i, slot):
        return pltpu.make_async_copy(
            obuf.at[slot], o_hbm.at[pl.ds(i * CHUNK, CHUNK), :], out_sem.at[slot]
        )

    for k in range(NBUF):
        get(k, k).start()

    def step(i, _):
        slot = jax.lax.rem(i, NBUF)
        get(i, slot).wait()

        @pl.when(i >= NBUF)
        def _():
            put(i - NBUF, slot).wait()

        obuf[slot] = _gelu(xbuf[slot])
        put(i, slot).start()

        @pl.when(i + NBUF < nchunks)
        def _():
            get(i + NBUF, slot).start()

        return 0

    jax.lax.fori_loop(0, nchunks, step, 0)

    for k in range(NBUF):
        last = nchunks - NBUF + k
        put(last, jax.lax.rem(jnp.int32(last), NBUF)).wait()


def kernel(x, log_tau, log_blend):
    b, t, d = x.shape
    rows = b * t
    x2 = x.reshape(rows, d)
    out = pl.pallas_call(
        _pipeline,
        in_specs=[pl.BlockSpec(memory_space=pl.ANY)],
        out_specs=pl.BlockSpec(memory_space=pl.ANY),
        out_shape=jax.ShapeDtypeStruct((rows, d), x.dtype),
        scratch_shapes=[
            pltpu.VMEM((NBUF, CHUNK, d), jnp.float32),
            pltpu.VMEM((NBUF, CHUNK, d), jnp.float32),
            pltpu.SemaphoreType.DMA((NBUF,)),
            pltpu.SemaphoreType.DMA((NBUF,)),
        ],
    )(x2)
    return out.reshape(b, t, d)
